# Initial kernel scaffold; baseline (speedup 1.0000x reference)
import functools

import jax
import jax.numpy as jnp
from jax.experimental import pallas as pl
from jax.experimental.pallas import tpu as pltpu

_N = 10000
_E = 320000
_K1, _K2, _K3 = 2000, 1000, 500


def _mm_acc_kernel(a_ref, b_ref, o_ref):
    @pl.when(pl.program_id(0) == 0)
    def _():
        o_ref[...] = jnp.zeros_like(o_ref)

    o_ref[...] += jnp.dot(a_ref[...], b_ref[...],
                          preferred_element_type=jnp.float32)


def _pallas_matmul_bf16(a, b, k_block):
    m, k = a.shape
    _, n = b.shape
    steps = k // k_block
    return pl.pallas_call(
        _mm_acc_kernel,
        grid=(steps,),
        in_specs=[
            pl.BlockSpec((m, k_block), lambda i: (0, i)),
            pl.BlockSpec((k_block, n), lambda i: (i, 0)),
        ],
        out_specs=pl.BlockSpec((m, n), lambda i: (0, 0)),
        out_shape=jax.ShapeDtypeStruct((m, n), jnp.float32),
    )(a.astype(jnp.bfloat16), b.astype(jnp.bfloat16))


def _gcn_sparse(x, src, dst, n, W, b):
    h = x @ W
    deg = jnp.zeros((n,), jnp.float32).at[dst].add(1.0) + 1.0
    dis = jax.lax.rsqrt(deg)
    msg = h[src] * (dis[src] * dis[dst])[:, None]
    out = jnp.zeros_like(h).at[dst].add(msg)
    out = out + h * (dis * dis)[:, None]
    return out + b


def _gcn_dense(x, M, W, b):
    n = M.shape[0]
    h = x @ W
    idx = jnp.arange(n)
    Mh = M.at[idx, idx].add(1.0)
    deg = jnp.sum(Mh, axis=1)
    dis = jax.lax.rsqrt(jnp.maximum(deg, 1e-12))
    Mn = Mh * dis[:, None] * dis[None, :]
    return Mn @ h + b


def _topk_pool(x, p, k):
    score = jnp.tanh((x @ p) / jnp.linalg.norm(p))
    sval, perm = jax.lax.top_k(score, k)
    return x[perm] * sval[:, None], perm


def _augment_pool_sparse(src, dst, perm, n, k):
    pos = jnp.full((n,), k, jnp.int32).at[perm].set(jnp.arange(k, dtype=jnp.int32))
    ak = jnp.arange(k)
    Br = jnp.zeros((k, n), jnp.float32).at[pos[dst], src].add(1.0, mode='drop')
    Br = Br.at[ak, perm].add(1.0)
    Bc = jnp.zeros((n, k), jnp.float32).at[dst, pos[src]].add(1.0, mode='drop')
    Bc = Bc.at[perm, ak].add(1.0)
    M2 = _pallas_matmul_bf16(Br, Bc, k_block=2000)
    return M2.at[ak, ak].set(0.0)


def _augment_pool_dense(M, perm):
    n = M.shape[0]
    idx = jnp.arange(n)
    B = M.at[idx, idx].add(1.0)
    M2 = B[perm, :] @ B[:, perm]
    k = perm.shape[0]
    ak = jnp.arange(k)
    return M2.at[ak, ak].set(0.0)


def kernel(x, edge_index, W_d0, b_d0, p1, W_d1, b_d1, p2, W_d2, b_d2, p3,
           W_d3, b_d3, W_u0, b_u0, W_u1, b_u1, W_u2, b_u2):
    src, dst = edge_index[0], edge_index[1]
    x1 = jax.nn.relu(_gcn_sparse(x, src, dst, _N, W_d0, b_d0))
    x1p, perm1 = _topk_pool(x1, p1, _K1)
    M1 = _augment_pool_sparse(src, dst, perm1, _N, _K1)
    x2 = jax.nn.relu(_gcn_dense(x1p, M1, W_d1, b_d1))
    x2p, perm2 = _topk_pool(x2, p2, _K2)
    M2 = _augment_pool_dense(M1, perm2)
    x3 = jax.nn.relu(_gcn_dense(x2p, M2, W_d2, b_d2))
    x3p, perm3 = _topk_pool(x3, p3, _K3)
    M3 = _augment_pool_dense(M2, perm3)
    x4 = jax.nn.relu(_gcn_dense(x3p, M3, W_d3, b_d3))
    h = x3 + jnp.zeros_like(x3).at[perm3].set(x4)
    h = jax.nn.relu(_gcn_dense(h, M2, W_u0, b_u0))
    h = x2 + jnp.zeros_like(x2).at[perm2].set(h)
    h = jax.nn.relu(_gcn_dense(h, M1, W_u1, b_u1))
    h = x1 + jnp.zeros_like(x1).at[perm1].set(h)
    h = _gcn_sparse(h, src, dst, _N, W_u2, b_u2)
    return jax.nn.log_softmax(h, axis=1)


# speedup vs baseline: 1.0013x; 1.0013x over previous
import functools

import jax
import jax.numpy as jnp
from jax.experimental import pallas as pl
from jax.experimental.pallas import tpu as pltpu

_N = 10000
_E = 320000
_K1, _K2, _K3 = 2000, 1000, 500


def _mm_acc_kernel(a_ref, b_ref, o_ref):
    @pl.when(pl.program_id(0) == 0)
    def _():
        o_ref[...] = jnp.zeros_like(o_ref)

    o_ref[...] += jnp.dot(a_ref[...], b_ref[...],
                          preferred_element_type=jnp.float32)


def _pallas_matmul_bf16(a, b, k_block):
    m, k = a.shape
    _, n = b.shape
    steps = k // k_block
    return pl.pallas_call(
        _mm_acc_kernel,
        grid=(steps,),
        in_specs=[
            pl.BlockSpec((m, k_block), lambda i: (0, i)),
            pl.BlockSpec((k_block, n), lambda i: (i, 0)),
        ],
        out_specs=pl.BlockSpec((m, n), lambda i: (0, 0)),
        out_shape=jax.ShapeDtypeStruct((m, n), jnp.float32),
    )(a.astype(jnp.bfloat16), b.astype(jnp.bfloat16))


def _gcn_sparse(x, src, dst, n, W, b):
    h = x @ W
    deg = jnp.zeros((n,), jnp.float32).at[dst].add(1.0) + 1.0
    dis = jax.lax.rsqrt(deg)
    msg = h[src] * (dis[src] * dis[dst])[:, None]
    out = jnp.zeros_like(h).at[dst].add(msg)
    out = out + h * (dis * dis)[:, None]
    return out + b


def _gcn_dense(x, M, W, b):
    n = M.shape[0]
    h = x @ W
    idx = jnp.arange(n)
    Mh = M.at[idx, idx].add(1.0)
    deg = jnp.sum(Mh, axis=1)
    dis = jax.lax.rsqrt(jnp.maximum(deg, 1e-12))
    Mn = Mh * dis[:, None] * dis[None, :]
    return Mn @ h + b


def _topk_pool(x, p, k):
    score = jnp.tanh((x @ p) / jnp.linalg.norm(p))
    sval, perm = jax.lax.top_k(score, k)
    return x[perm] * sval[:, None], perm


def _augment_pool_sparse(src, dst, perm, n, k):
    n_pad = 10240
    pos = jnp.full((n,), k, jnp.int32).at[perm].set(jnp.arange(k, dtype=jnp.int32))
    ak = jnp.arange(k)
    Br = jnp.zeros((k, n_pad), jnp.float32).at[pos[dst], src].add(1.0, mode='drop')
    Br = Br.at[ak, perm].add(1.0)
    Bc = jnp.zeros((n_pad, k), jnp.float32).at[dst, pos[src]].add(1.0, mode='drop')
    Bc = Bc.at[perm, ak].add(1.0)
    M2 = _pallas_matmul_bf16(Br, Bc, k_block=1024)
    return M2.at[ak, ak].set(0.0)


def _augment_pool_dense(M, perm):
    n = M.shape[0]
    idx = jnp.arange(n)
    B = M.at[idx, idx].add(1.0)
    M2 = B[perm, :] @ B[:, perm]
    k = perm.shape[0]
    ak = jnp.arange(k)
    return M2.at[ak, ak].set(0.0)


def kernel(x, edge_index, W_d0, b_d0, p1, W_d1, b_d1, p2, W_d2, b_d2, p3,
           W_d3, b_d3, W_u0, b_u0, W_u1, b_u1, W_u2, b_u2):
    src, dst = edge_index[0], edge_index[1]
    x1 = jax.nn.relu(_gcn_sparse(x, src, dst, _N, W_d0, b_d0))
    x1p, perm1 = _topk_pool(x1, p1, _K1)
    M1 = _augment_pool_sparse(src, dst, perm1, _N, _K1)
    x2 = jax.nn.relu(_gcn_dense(x1p, M1, W_d1, b_d1))
    x2p, perm2 = _topk_pool(x2, p2, _K2)
    M2 = _augment_pool_dense(M1, perm2)
    x3 = jax.nn.relu(_gcn_dense(x2p, M2, W_d2, b_d2))
    x3p, perm3 = _topk_pool(x3, p3, _K3)
    M3 = _augment_pool_dense(M2, perm3)
    x4 = jax.nn.relu(_gcn_dense(x3p, M3, W_d3, b_d3))
    h = x3 + jnp.zeros_like(x3).at[perm3].set(x4)
    h = jax.nn.relu(_gcn_dense(h, M2, W_u0, b_u0))
    h = x2 + jnp.zeros_like(x2).at[perm2].set(h)
    h = jax.nn.relu(_gcn_dense(h, M1, W_u1, b_u1))
    h = x1 + jnp.zeros_like(x1).at[perm1].set(h)
    h = _gcn_sparse(h, src, dst, _N, W_u2, b_u2)
    return jax.nn.log_softmax(h, axis=1)


# speedup vs baseline: 1.4876x; 1.4857x over previous
import functools

import jax
import jax.numpy as jnp
from jax.experimental import pallas as pl
from jax.experimental.pallas import tpu as pltpu

_N = 10000
_E = 320000
_K1, _K2, _K3 = 2000, 1000, 500


def _mm_acc_kernel(a_ref, b_ref, o_ref):
    @pl.when(pl.program_id(0) == 0)
    def _():
        o_ref[...] = jnp.zeros_like(o_ref)

    o_ref[...] += jnp.dot(a_ref[...], b_ref[...],
                          preferred_element_type=jnp.float32)


def _pallas_matmul_bf16(a, b, k_block):
    m, k = a.shape
    _, n = b.shape
    steps = k // k_block
    return pl.pallas_call(
        _mm_acc_kernel,
        grid=(steps,),
        in_specs=[
            pl.BlockSpec((m, k_block), lambda i: (0, i)),
            pl.BlockSpec((k_block, n), lambda i: (i, 0)),
        ],
        out_specs=pl.BlockSpec((m, n), lambda i: (0, 0)),
        out_shape=jax.ShapeDtypeStruct((m, n), jnp.float32),
    )(a.astype(jnp.bfloat16), b.astype(jnp.bfloat16))


def _gcn_sparse(x, src, dst, n, W, b, dis):
    h = x @ W
    hd = h * dis[:, None]
    agg = jnp.zeros_like(h).at[dst].add(hd[src])
    return (agg + hd) * dis[:, None] + b


def _gcn_dense(x, M, W, b):
    n = M.shape[0]
    h = x @ W
    idx = jnp.arange(n)
    Mh = M.at[idx, idx].add(1.0)
    deg = jnp.sum(Mh, axis=1)
    dis = jax.lax.rsqrt(jnp.maximum(deg, 1e-12))
    Mn = Mh * dis[:, None] * dis[None, :]
    return Mn @ h + b


def _topk_pool(x, p, k):
    score = jnp.tanh((x @ p) / jnp.linalg.norm(p))
    sval, perm = jax.lax.top_k(score, k)
    return x[perm] * sval[:, None], perm


def _augment_pool_sparse(src, dst, perm, n, k):
    n_pad = 10240
    pos = jnp.full((n,), k, jnp.int32).at[perm].set(jnp.arange(k, dtype=jnp.int32))
    ak = jnp.arange(k)
    Br = jnp.zeros((k, n_pad), jnp.float32).at[pos[dst], src].add(1.0, mode='drop')
    Br = Br.at[ak, perm].add(1.0)
    Bc = jnp.zeros((n_pad, k), jnp.float32).at[dst, pos[src]].add(1.0, mode='drop')
    Bc = Bc.at[perm, ak].add(1.0)
    M2 = _pallas_matmul_bf16(Br, Bc, k_block=1024)
    return M2.at[ak, ak].set(0.0)


def _augment_pool_dense(M, perm):
    n = M.shape[0]
    idx = jnp.arange(n)
    B = M.at[idx, idx].add(1.0)
    M2 = B[perm, :] @ B[:, perm]
    k = perm.shape[0]
    ak = jnp.arange(k)
    return M2.at[ak, ak].set(0.0)


def kernel(x, edge_index, W_d0, b_d0, p1, W_d1, b_d1, p2, W_d2, b_d2, p3,
           W_d3, b_d3, W_u0, b_u0, W_u1, b_u1, W_u2, b_u2):
    src, dst = edge_index[0], edge_index[1]
    deg = jnp.zeros((_N,), jnp.float32).at[dst].add(1.0) + 1.0
    dis = jax.lax.rsqrt(deg)
    x1 = jax.nn.relu(_gcn_sparse(x, src, dst, _N, W_d0, b_d0, dis))
    x1p, perm1 = _topk_pool(x1, p1, _K1)
    M1 = _augment_pool_sparse(src, dst, perm1, _N, _K1)
    x2 = jax.nn.relu(_gcn_dense(x1p, M1, W_d1, b_d1))
    x2p, perm2 = _topk_pool(x2, p2, _K2)
    M2 = _augment_pool_dense(M1, perm2)
    x3 = jax.nn.relu(_gcn_dense(x2p, M2, W_d2, b_d2))
    x3p, perm3 = _topk_pool(x3, p3, _K3)
    M3 = _augment_pool_dense(M2, perm3)
    x4 = jax.nn.relu(_gcn_dense(x3p, M3, W_d3, b_d3))
    h = x3 + jnp.zeros_like(x3).at[perm3].set(x4)
    h = jax.nn.relu(_gcn_dense(h, M2, W_u0, b_u0))
    h = x2 + jnp.zeros_like(x2).at[perm2].set(h)
    h = jax.nn.relu(_gcn_dense(h, M1, W_u1, b_u1))
    h = x1 + jnp.zeros_like(x1).at[perm1].set(h)
    h = _gcn_sparse(h, src, dst, _N, W_u2, b_u2, dis)
    return jax.nn.log_softmax(h, axis=1)


# speedup vs baseline: 2.2179x; 1.4910x over previous
import functools

import jax
import jax.numpy as jnp
from jax import lax
from jax.experimental import pallas as pl
from jax.experimental.pallas import tpu as pltpu
from jax.experimental.pallas import tpu_sc as plsc

_N = 10000
_E = 320000
_K1, _K2, _K3 = 2000, 1000, 500
_NPAD = 10240

_NC = 2
_NS = 16
_NW = _NC * _NS


def _edge_agg_sc(hd, src, dst, zeros_pad):
    epw = _E // _NW
    ch = 2000
    rpt = _NPAD // _NS

    mesh = plsc.VectorSubcoreMesh(core_axis_name="c", subcore_axis_name="s",
                                  num_cores=_NC, num_subcores=_NS)

    @functools.partial(
        pl.kernel,
        out_type=jax.ShapeDtypeStruct((_NC, _NPAD, 32), jnp.float32),
        mesh=mesh,
        compiler_params=pltpu.CompilerParams(use_tc_tiling_on_sc=False),
        scratch_types=[
            pltpu.VMEM((ch,), jnp.int32),
            pltpu.VMEM((ch,), jnp.int32),
            pltpu.VMEM((ch, 32), jnp.float32),
            pltpu.VMEM_SHARED((_NPAD, 32), jnp.float32),
            pltpu.VMEM_SHARED((_NPAD, 32), jnp.float32),
            pltpu.SemaphoreType.DMA,
        ],
    )
    def k(hd_hbm, src_hbm, dst_hbm, z_hbm, out_hbm, sv, dv, rows, table, acc,
          sem):
        cid = lax.axis_index("c")
        sid = lax.axis_index("s")
        wid = sid * _NC + cid
        pltpu.sync_copy(hd_hbm.at[pl.ds(sid * rpt, rpt)],
                        table.at[pl.ds(sid * rpt, rpt)])
        pltpu.sync_copy(z_hbm.at[pl.ds(sid * rpt, rpt)],
                        acc.at[pl.ds(sid * rpt, rpt)])
        plsc.subcore_barrier()

        def chunk(ci, carry):
            off = wid * epw + ci * ch
            pltpu.sync_copy(src_hbm.at[pl.ds(off, ch)], sv)
            pltpu.sync_copy(dst_hbm.at[pl.ds(off, ch)], dv)
            pltpu.async_copy(table.at[sv], rows, sem).wait()
            pltpu.sync_copy(rows, acc.at[dv], add=True)
            return carry

        lax.fori_loop(0, epw // ch, chunk, 0)
        plsc.subcore_barrier()
        pltpu.sync_copy(acc.at[pl.ds(sid * rpt, rpt)],
                        out_hbm.at[cid, pl.ds(sid * rpt, rpt)])

    return k(hd, src, dst, zeros_pad)


def _mm_acc_kernel(a_ref, b_ref, o_ref):
    @pl.when(pl.program_id(0) == 0)
    def _():
        o_ref[...] = jnp.zeros_like(o_ref)

    o_ref[...] += jnp.dot(a_ref[...], b_ref[...],
                          preferred_element_type=jnp.float32)


def _pallas_matmul_bf16(a, b, k_block):
    m, k = a.shape
    _, n = b.shape
    steps = k // k_block
    return pl.pallas_call(
        _mm_acc_kernel,
        grid=(steps,),
        in_specs=[
            pl.BlockSpec((m, k_block), lambda i: (0, i)),
            pl.BlockSpec((k_block, n), lambda i: (i, 0)),
        ],
        out_specs=pl.BlockSpec((m, n), lambda i: (0, 0)),
        out_shape=jax.ShapeDtypeStruct((m, n), jnp.float32),
    )(a.astype(jnp.bfloat16), b.astype(jnp.bfloat16))


def _sc_aggregate(hd, n, src, dst, zeros_pad):
    hd_pad = jnp.zeros((_NPAD, 32), jnp.float32).at[:n, :].set(hd)
    parts = _edge_agg_sc(hd_pad, src, dst, zeros_pad)
    return (parts[0] + parts[1])[:n]


def _gcn_sparse(x, src, dst, n, W, b, dis, zeros_pad, matmul_first):
    if matmul_first:
        hd = (x @ W) * dis[:, None]
        agg = _sc_aggregate(hd, n, src, dst, zeros_pad)
        return (agg + hd) * dis[:, None] + b
    g = x * dis[:, None]
    agg = _sc_aggregate(g, n, src, dst, zeros_pad)
    return ((agg + g) @ W) * dis[:, None] + b


def _gcn_dense(x, M, W, b):
    n = M.shape[0]
    h = x @ W
    idx = jnp.arange(n)
    Mh = M.at[idx, idx].add(1.0)
    deg = jnp.sum(Mh, axis=1)
    dis = jax.lax.rsqrt(jnp.maximum(deg, 1e-12))
    Mn = Mh * dis[:, None] * dis[None, :]
    return Mn @ h + b


def _topk_pool(x, p, k):
    score = jnp.tanh((x @ p) / jnp.linalg.norm(p))
    sval, perm = jax.lax.top_k(score, k)
    return x[perm] * sval[:, None], perm


def _augment_pool_sparse(src, dst, perm, n, k):
    n_pad = 10240
    pos = jnp.full((n,), k, jnp.int32).at[perm].set(jnp.arange(k, dtype=jnp.int32))
    ak = jnp.arange(k)
    Br = jnp.zeros((k, n_pad), jnp.float32).at[pos[dst], src].add(1.0, mode='drop')
    Br = Br.at[ak, perm].add(1.0)
    Bc = jnp.zeros((n_pad, k), jnp.float32).at[dst, pos[src]].add(1.0, mode='drop')
    Bc = Bc.at[perm, ak].add(1.0)
    M2 = _pallas_matmul_bf16(Br, Bc, k_block=1024)
    return M2.at[ak, ak].set(0.0)


def _augment_pool_dense(M, perm):
    n = M.shape[0]
    idx = jnp.arange(n)
    B = M.at[idx, idx].add(1.0)
    M2 = B[perm, :] @ B[:, perm]
    k = perm.shape[0]
    ak = jnp.arange(k)
    return M2.at[ak, ak].set(0.0)


def kernel(x, edge_index, W_d0, b_d0, p1, W_d1, b_d1, p2, W_d2, b_d2, p3,
           W_d3, b_d3, W_u0, b_u0, W_u1, b_u1, W_u2, b_u2):
    src, dst = edge_index[0], edge_index[1]
    deg = jnp.zeros((_N,), jnp.float32).at[dst].add(1.0) + 1.0
    dis = jax.lax.rsqrt(deg)
    zeros_pad = jnp.zeros((_NPAD, 32), jnp.float32)
    x1 = jax.nn.relu(_gcn_sparse(x, src, dst, _N, W_d0, b_d0, dis, zeros_pad, True))
    x1p, perm1 = _topk_pool(x1, p1, _K1)
    M1 = _augment_pool_sparse(src, dst, perm1, _N, _K1)
    x2 = jax.nn.relu(_gcn_dense(x1p, M1, W_d1, b_d1))
    x2p, perm2 = _topk_pool(x2, p2, _K2)
    M2 = _augment_pool_dense(M1, perm2)
    x3 = jax.nn.relu(_gcn_dense(x2p, M2, W_d2, b_d2))
    x3p, perm3 = _topk_pool(x3, p3, _K3)
    M3 = _augment_pool_dense(M2, perm3)
    x4 = jax.nn.relu(_gcn_dense(x3p, M3, W_d3, b_d3))
    h = x3 + jnp.zeros_like(x3).at[perm3].set(x4)
    h = jax.nn.relu(_gcn_dense(h, M2, W_u0, b_u0))
    h = x2 + jnp.zeros_like(x2).at[perm2].set(h)
    h = jax.nn.relu(_gcn_dense(h, M1, W_u1, b_u1))
    h = x1 + jnp.zeros_like(x1).at[perm1].set(h)
    h = _gcn_sparse(h, src, dst, _N, W_u2, b_u2, dis, zeros_pad, False)
    return jax.nn.log_softmax(h, axis=1)


# speedup vs baseline: 6.6199x; 2.9848x over previous
import functools

import jax
import jax.numpy as jnp
from jax import lax
from jax.experimental import pallas as pl
from jax.experimental.pallas import tpu as pltpu
from jax.experimental.pallas import tpu_sc as plsc

_N = 10000
_E = 320000
_K1, _K2, _K3 = 2000, 1000, 500
_NPAD = 10240

_NC = 2
_NS = 16
_NW = _NC * _NS


def _edge_agg_sc(hd, src, dst, zeros_pad):
    epw = _E // _NW
    ch = 2000
    rpt = _NPAD // _NS

    mesh = plsc.VectorSubcoreMesh(core_axis_name="c", subcore_axis_name="s",
                                  num_cores=_NC, num_subcores=_NS)

    @functools.partial(
        pl.kernel,
        out_type=jax.ShapeDtypeStruct((_NC, _NPAD, 32), jnp.float32),
        mesh=mesh,
        compiler_params=pltpu.CompilerParams(use_tc_tiling_on_sc=False),
        scratch_types=[
            pltpu.VMEM((ch,), jnp.int32),
            pltpu.VMEM((ch,), jnp.int32),
            pltpu.VMEM((ch, 32), jnp.float32),
            pltpu.VMEM_SHARED((_NPAD, 32), jnp.float32),
            pltpu.VMEM_SHARED((_NPAD, 32), jnp.float32),
            pltpu.SemaphoreType.DMA,
        ],
    )
    def k(hd_hbm, src_hbm, dst_hbm, z_hbm, out_hbm, sv, dv, rows, table, acc,
          sem):
        cid = lax.axis_index("c")
        sid = lax.axis_index("s")
        wid = sid * _NC + cid
        pltpu.sync_copy(hd_hbm.at[pl.ds(sid * rpt, rpt)],
                        table.at[pl.ds(sid * rpt, rpt)])
        pltpu.sync_copy(z_hbm.at[pl.ds(sid * rpt, rpt)],
                        acc.at[pl.ds(sid * rpt, rpt)])
        plsc.subcore_barrier()

        def chunk(ci, carry):
            off = wid * epw + ci * ch
            pltpu.sync_copy(src_hbm.at[pl.ds(off, ch)], sv)
            pltpu.sync_copy(dst_hbm.at[pl.ds(off, ch)], dv)
            pltpu.async_copy(table.at[sv], rows, sem).wait()
            pltpu.sync_copy(rows, acc.at[dv], add=True)
            return carry

        lax.fori_loop(0, epw // ch, chunk, 0)
        plsc.subcore_barrier()
        pltpu.sync_copy(acc.at[pl.ds(sid * rpt, rpt)],
                        out_hbm.at[cid, pl.ds(sid * rpt, rpt)])

    return k(hd, src, dst, zeros_pad)


def _mm_acc_kernel(a_ref, b_ref, o_ref):
    @pl.when(pl.program_id(0) == 0)
    def _():
        o_ref[...] = jnp.zeros_like(o_ref)

    o_ref[...] += jnp.dot(a_ref[...], b_ref[...],
                          preferred_element_type=jnp.float32)


def _pallas_matmul_bf16(a, b, k_block):
    m, k = a.shape
    _, n = b.shape
    steps = k // k_block
    return pl.pallas_call(
        _mm_acc_kernel,
        grid=(steps,),
        in_specs=[
            pl.BlockSpec((m, k_block), lambda i: (0, i)),
            pl.BlockSpec((k_block, n), lambda i: (i, 0)),
        ],
        out_specs=pl.BlockSpec((m, n), lambda i: (0, 0)),
        out_shape=jax.ShapeDtypeStruct((m, n), jnp.float32),
    )(a.astype(jnp.bfloat16), b.astype(jnp.bfloat16))


def _sc_aggregate(hd, n, src, dst, zeros_pad):
    hd_pad = jnp.zeros((_NPAD, 32), jnp.float32).at[:n, :].set(hd)
    parts = _edge_agg_sc(hd_pad, src, dst, zeros_pad)
    return (parts[0] + parts[1])[:n]


def _pos_lookup_sc(pos_pad, src, dst):
    epw = _E // _NW
    ch = 2000

    mesh = plsc.VectorSubcoreMesh(core_axis_name="c", subcore_axis_name="s",
                                  num_cores=_NC, num_subcores=_NS)

    @functools.partial(
        pl.kernel,
        out_type=(jax.ShapeDtypeStruct((_E,), jnp.int32),
                  jax.ShapeDtypeStruct((_E,), jnp.int32)),
        mesh=mesh,
        compiler_params=pltpu.CompilerParams(use_tc_tiling_on_sc=False,
                                             needs_layout_passes=False),
        scratch_types=[
            pltpu.VMEM((_NPAD,), jnp.int32),
            pltpu.VMEM((ch,), jnp.int32),
            pltpu.VMEM((ch,), jnp.int32),
        ],
    )
    def k(pos_hbm, src_hbm, dst_hbm, ps_hbm, pd_hbm, table, iv, ov):
        cid = lax.axis_index("c")
        sid = lax.axis_index("s")
        wid = sid * _NC + cid
        pltpu.sync_copy(pos_hbm, table)

        def do_stream(idx_hbm, out_hbm):
            def chunk(ci, carry):
                off = wid * epw + ci * ch
                pltpu.sync_copy(idx_hbm.at[pl.ds(off, ch)], iv)

                def lanes(j, c2):
                    idx = iv[pl.ds(j * 16, 16)]
                    ov[pl.ds(j * 16, 16)] = plsc.load_gather(table, [idx])
                    return c2

                lax.fori_loop(0, ch // 16, lanes, 0)
                pltpu.sync_copy(ov, out_hbm.at[pl.ds(off, ch)])
                return carry

            lax.fori_loop(0, epw // ch, chunk, 0)

        do_stream(src_hbm, ps_hbm)
        do_stream(dst_hbm, pd_hbm)

    return k(pos_pad, src, dst)


def _gcn_sparse(x, src, dst, n, W, b, dis, zeros_pad, matmul_first):
    if matmul_first:
        hd = (x @ W) * dis[:, None]
        agg = _sc_aggregate(hd, n, src, dst, zeros_pad)
        return (agg + hd) * dis[:, None] + b
    g = x * dis[:, None]
    agg = _sc_aggregate(g, n, src, dst, zeros_pad)
    return ((agg + g) @ W) * dis[:, None] + b


def _gcn_dense(x, M, W, b):
    n = M.shape[0]
    h = x @ W
    idx = jnp.arange(n)
    Mh = M.at[idx, idx].add(1.0)
    deg = jnp.sum(Mh, axis=1)
    dis = jax.lax.rsqrt(jnp.maximum(deg, 1e-12))
    Mn = Mh * dis[:, None] * dis[None, :]
    return Mn @ h + b


def _topk_pool(x, p, k):
    score = jnp.tanh((x @ p) / jnp.linalg.norm(p))
    sval, perm = jax.lax.top_k(score, k)
    return x[perm] * sval[:, None], perm


def _augment_pool_sparse(src, dst, perm, n, k):
    pos = jnp.full((_NPAD,), k, jnp.int32).at[perm].set(jnp.arange(k, dtype=jnp.int32))
    ps, pd = _pos_lookup_sc(pos, src, dst)
    ak = jnp.arange(k)
    Br = jnp.zeros((k, _NPAD), jnp.float32).at[pd, src].add(1.0, mode='drop')
    Br = Br.at[ak, perm].add(1.0)
    Bc = jnp.zeros((_NPAD, k), jnp.float32).at[dst, ps].add(1.0, mode='drop')
    Bc = Bc.at[perm, ak].add(1.0)
    M2 = _pallas_matmul_bf16(Br, Bc, k_block=1024)
    return M2.at[ak, ak].set(0.0)


def _augment_pool_dense(M, perm):
    n = M.shape[0]
    idx = jnp.arange(n)
    B = M.at[idx, idx].add(1.0)
    M2 = B[perm, :] @ B[:, perm]
    k = perm.shape[0]
    ak = jnp.arange(k)
    return M2.at[ak, ak].set(0.0)


def kernel(x, edge_index, W_d0, b_d0, p1, W_d1, b_d1, p2, W_d2, b_d2, p3,
           W_d3, b_d3, W_u0, b_u0, W_u1, b_u1, W_u2, b_u2):
    src, dst = edge_index[0], edge_index[1]
    deg = jnp.zeros((_N,), jnp.float32).at[dst].add(1.0) + 1.0
    dis = jax.lax.rsqrt(deg)
    zeros_pad = jnp.zeros((_NPAD, 32), jnp.float32)
    x1 = jax.nn.relu(_gcn_sparse(x, src, dst, _N, W_d0, b_d0, dis, zeros_pad, True))
    x1p, perm1 = _topk_pool(x1, p1, _K1)
    M1 = _augment_pool_sparse(src, dst, perm1, _N, _K1)
    x2 = jax.nn.relu(_gcn_dense(x1p, M1, W_d1, b_d1))
    x2p, perm2 = _topk_pool(x2, p2, _K2)
    M2 = _augment_pool_dense(M1, perm2)
    x3 = jax.nn.relu(_gcn_dense(x2p, M2, W_d2, b_d2))
    x3p, perm3 = _topk_pool(x3, p3, _K3)
    M3 = _augment_pool_dense(M2, perm3)
    x4 = jax.nn.relu(_gcn_dense(x3p, M3, W_d3, b_d3))
    h = x3 + jnp.zeros_like(x3).at[perm3].set(x4)
    h = jax.nn.relu(_gcn_dense(h, M2, W_u0, b_u0))
    h = x2 + jnp.zeros_like(x2).at[perm2].set(h)
    h = jax.nn.relu(_gcn_dense(h, M1, W_u1, b_u1))
    h = x1 + jnp.zeros_like(x1).at[perm1].set(h)
    h = _gcn_sparse(h, src, dst, _N, W_u2, b_u2, dis, zeros_pad, False)
    return jax.nn.log_softmax(h, axis=1)
